# baseline (speedup 1.0000x reference)
import functools

import jax
import jax.numpy as jnp
from jax import lax
from jax.experimental import pallas as pl
from jax.experimental.pallas import tpu as pltpu
from jax.experimental.pallas import tpu_sc as plsc

B, T, D = 16, 1024, 32
DP = 128
K = 8192
KT = 512
KSTEPS = K // KT
CHUNK = 4096
TILES_PER_CHUNK = CHUNK // KT
IDX_CHUNK = 128


def _argmin_body(zn_ref, en_ref, a_ref, c_ref, out_ref, cm_ref, ci_ref, mn_ref, ix_ref):
    k = pl.program_id(1)
    znb = zn_ref[0]
    enb = en_ref[...]
    s = lax.dot_general(
        enb, znb, (((1,), (1,)), ((), ())),
        preferred_element_type=jnp.float32,
        precision=lax.Precision.DEFAULT,
    )
    dist = (a_ref[0] + c_ref[...]) - 2.0 * s
    m = jnp.min(dist, axis=0, keepdims=True)
    ids = lax.broadcasted_iota(jnp.int32, dist.shape, 0) + k * KT
    cand = jnp.min(
        jnp.where(dist == m, ids, jnp.int32(2**31 - 1)), axis=0, keepdims=True
    )

    @pl.when(k % TILES_PER_CHUNK == 0)
    def _():
        cm_ref[...] = m
        ci_ref[...] = cand

    @pl.when(k % TILES_PER_CHUNK != 0)
    def _():
        upd = m < cm_ref[...]
        cm_ref[...] = jnp.where(upd, m, cm_ref[...])
        ci_ref[...] = jnp.where(upd, cand, ci_ref[...])

    @pl.when(k % TILES_PER_CHUNK == TILES_PER_CHUNK - 1)
    def _():
        cmb = cm_ref[...].astype(jnp.bfloat16).astype(jnp.float32)

        @pl.when(k == TILES_PER_CHUNK - 1)
        def _():
            mn_ref[...] = cmb
            ix_ref[...] = ci_ref[...]

        @pl.when(k > TILES_PER_CHUNK - 1)
        def _():
            win = mn_ref[...] > cm_ref[...]
            mn_ref[...] = jnp.where(win, cmb, mn_ref[...])
            ix_ref[...] = jnp.where(win, ci_ref[...], ix_ref[...])

    @pl.when(k == KSTEPS - 1)
    def _():
        out_ref[0] = ix_ref[...]


def _argmin_call(zn3, en, a3, c2):
    return pl.pallas_call(
        _argmin_body,
        grid=(B, KSTEPS),
        in_specs=[
            pl.BlockSpec((1, T, D), lambda b, k: (b, 0, 0)),
            pl.BlockSpec((KT, D), lambda b, k: (k, 0)),
            pl.BlockSpec((1, 1, T), lambda b, k: (b, 0, 0)),
            pl.BlockSpec((KT, 1), lambda b, k: (k, 0)),
        ],
        out_specs=pl.BlockSpec((1, 1, T), lambda b, k: (b, 0, 0)),
        out_shape=jax.ShapeDtypeStruct((B, 1, T), jnp.int32),
        scratch_shapes=[
            pltpu.VMEM((1, T), jnp.float32),
            pltpu.VMEM((1, T), jnp.int32),
            pltpu.VMEM((1, T), jnp.float32),
            pltpu.VMEM((1, T), jnp.int32),
        ],
        compiler_params=pltpu.CompilerParams(
            dimension_semantics=("arbitrary", "arbitrary"),
        ),
    )(zn3, en, a3, c2)


def _finalize_body(zq_ref, zn_ref, out_ref, loss_ref, acc_ref):
    b = pl.program_id(0)
    zq = zq_ref[0][:, :D]
    znb = zn_ref[0]
    n = jnp.sqrt(jnp.sum(zq * zq, axis=0, keepdims=True))
    zqn = zq / jnp.maximum(n, 1e-12)
    out_ref[0] = znb + (zqn - znb)
    part = jnp.sum((zqn - znb) ** 2)

    @pl.when(b == 0)
    def _():
        acc_ref[0, 0] = part

    @pl.when(b > 0)
    def _():
        acc_ref[0, 0] = acc_ref[0, 0] + part

    @pl.when(b == B - 1)
    def _():
        m = acc_ref[0, 0] / (B * T * D)
        loss_ref[...] = jnp.full((1, 1), m + 0.25 * m, jnp.float32)


def _finalize_call(zq3, zn3):
    return pl.pallas_call(
        _finalize_body,
        grid=(B,),
        in_specs=[
            pl.BlockSpec((1, T, DP), lambda b: (b, 0, 0)),
            pl.BlockSpec((1, T, D), lambda b: (b, 0, 0)),
        ],
        out_specs=[
            pl.BlockSpec((1, T, D), lambda b: (b, 0, 0)),
            pl.BlockSpec((1, 1), lambda b: (0, 0)),
        ],
        out_shape=[
            jax.ShapeDtypeStruct((B, T, D), jnp.float32),
            jax.ShapeDtypeStruct((1, 1), jnp.float32),
        ],
        scratch_shapes=[pltpu.SMEM((1, 1), jnp.float32)],
        compiler_params=pltpu.CompilerParams(
            dimension_semantics=("arbitrary",),
        ),
    )(zq3, zn3)


def _sc_gather(Wp, idx):
    info = plsc.get_sparse_core_info()
    nc, ns = info.num_cores, info.num_subcores
    nw = nc * ns
    n = idx.shape[0]
    b_per_w = n // nw
    chunks = b_per_w // IDX_CHUNK
    mesh = plsc.VectorSubcoreMesh(core_axis_name="c", subcore_axis_name="s")

    @functools.partial(
        pl.kernel,
        mesh=mesh,
        out_type=jax.ShapeDtypeStruct((n, DP), jnp.float32),
        scratch_types=[
            pltpu.VMEM((IDX_CHUNK,), jnp.int32),
            pltpu.VMEM((IDX_CHUNK, DP), jnp.float32),
            pltpu.SemaphoreType.DMA,
        ],
    )
    def gather_kernel(table_hbm, idx_hbm, out_hbm, idx_v, rows_v, sem):
        wid = lax.axis_index("s") * nc + lax.axis_index("c")
        base = wid * b_per_w
        for j in range(chunks):
            off = base + j * IDX_CHUNK
            pltpu.sync_copy(idx_hbm.at[pl.ds(off, IDX_CHUNK)], idx_v)
            pltpu.async_copy(table_hbm.at[idx_v], rows_v, sem).wait()
            pltpu.sync_copy(rows_v, out_hbm.at[pl.ds(off, IDX_CHUNK)])

    return gather_kernel(Wp, idx)


def kernel(z, W):
    nz = jnp.linalg.norm(z, ord=2, axis=1, keepdims=True)
    zn3 = z / jnp.maximum(nz, 1e-12)
    nw_ = jnp.linalg.norm(W, ord=2, axis=1, keepdims=True)
    en = W / jnp.maximum(nw_, 1e-12)
    z_flat = zn3.reshape(B * T, D)
    a = jnp.sum(z_flat ** 2, axis=1, keepdims=True)
    a3 = a.reshape(B, T)[:, None, :]
    c2 = jnp.sum(W ** 2, axis=1).reshape(K, 1)

    idx3 = _argmin_call(zn3, en, a3, c2)
    idx = idx3.reshape(B * T)

    Wp = jnp.pad(W, ((0, 0), (0, DP - D)))
    rows = _sc_gather(Wp, idx)
    zq3 = rows.reshape(B, T, DP)

    zq_out, loss11 = _finalize_call(zq3, zn3)
    return zq_out, idx, loss11[0, 0]


# speedup vs baseline: 1.2618x; 1.2618x over previous
import functools

import jax
import jax.numpy as jnp
from jax import lax
from jax.experimental import pallas as pl
from jax.experimental.pallas import tpu as pltpu
from jax.experimental.pallas import tpu_sc as plsc

B, T, D = 16, 1024, 32
DP = 128
K = 8192
KT = 2048
KSTEPS = K // KT
CHUNK = 4096
TILES_PER_CHUNK = CHUNK // KT
IDX_CHUNK = 128


def _argmin_body(zn_ref, en_ref, a_ref, c_ref, out_ref, cm_ref, ci_ref, mn_ref, ix_ref):
    k = pl.program_id(1)
    znb = zn_ref[0]
    enb = en_ref[...]
    s2 = lax.dot_general(
        enb, znb, (((1,), (1,)), ((), ())),
        preferred_element_type=jnp.float32,
    )
    dist = (a_ref[0] + c_ref[...]) - s2
    m = jnp.min(dist, axis=0, keepdims=True)
    ids = lax.broadcasted_iota(jnp.int32, dist.shape, 0)
    cand = jnp.min(
        jnp.where(dist == m, ids, jnp.int32(2**31 - 1)), axis=0, keepdims=True
    ) + k * KT

    @pl.when(k % TILES_PER_CHUNK == 0)
    def _():
        cm_ref[...] = m
        ci_ref[...] = cand

    @pl.when(k % TILES_PER_CHUNK != 0)
    def _():
        upd = m < cm_ref[...]
        cm_ref[...] = jnp.where(upd, m, cm_ref[...])
        ci_ref[...] = jnp.where(upd, cand, ci_ref[...])

    @pl.when(k % TILES_PER_CHUNK == TILES_PER_CHUNK - 1)
    def _():
        cmb = cm_ref[...].astype(jnp.bfloat16).astype(jnp.float32)

        @pl.when(k == TILES_PER_CHUNK - 1)
        def _():
            mn_ref[...] = cmb
            ix_ref[...] = ci_ref[...]

        @pl.when(k > TILES_PER_CHUNK - 1)
        def _():
            win = mn_ref[...] > cm_ref[...]
            mn_ref[...] = jnp.where(win, cmb, mn_ref[...])
            ix_ref[...] = jnp.where(win, ci_ref[...], ix_ref[...])

    @pl.when(k == KSTEPS - 1)
    def _():
        out_ref[0] = ix_ref[...]


def _argmin_call(zn3, en, a3, c2):
    return pl.pallas_call(
        _argmin_body,
        grid=(B, KSTEPS),
        in_specs=[
            pl.BlockSpec((1, T, D), lambda b, k: (b, 0, 0)),
            pl.BlockSpec((KT, D), lambda b, k: (k, 0)),
            pl.BlockSpec((1, 1, T), lambda b, k: (b, 0, 0)),
            pl.BlockSpec((KT, 1), lambda b, k: (k, 0)),
        ],
        out_specs=pl.BlockSpec((1, 1, T), lambda b, k: (b, 0, 0)),
        out_shape=jax.ShapeDtypeStruct((B, 1, T), jnp.int32),
        scratch_shapes=[
            pltpu.VMEM((1, T), jnp.float32),
            pltpu.VMEM((1, T), jnp.int32),
            pltpu.VMEM((1, T), jnp.float32),
            pltpu.VMEM((1, T), jnp.int32),
        ],
        compiler_params=pltpu.CompilerParams(
            dimension_semantics=("arbitrary", "arbitrary"),
        ),
    )(zn3, en, a3, c2)


def _finalize_body(zq_ref, zn_ref, out_ref, loss_ref, acc_ref):
    b = pl.program_id(0)
    zq = zq_ref[0][:, :D]
    znb = zn_ref[0]
    n = jnp.sqrt(jnp.sum(zq * zq, axis=0, keepdims=True))
    zqn = zq / jnp.maximum(n, 1e-12)
    out_ref[0] = znb + (zqn - znb)
    part = jnp.sum((zqn - znb) ** 2)

    @pl.when(b == 0)
    def _():
        acc_ref[0, 0] = part

    @pl.when(b > 0)
    def _():
        acc_ref[0, 0] = acc_ref[0, 0] + part

    @pl.when(b == B - 1)
    def _():
        m = acc_ref[0, 0] / (B * T * D)
        loss_ref[...] = jnp.full((1, 1), m + 0.25 * m, jnp.float32)


def _finalize_call(zq3, zn3):
    return pl.pallas_call(
        _finalize_body,
        grid=(B,),
        in_specs=[
            pl.BlockSpec((1, T, DP), lambda b: (b, 0, 0)),
            pl.BlockSpec((1, T, D), lambda b: (b, 0, 0)),
        ],
        out_specs=[
            pl.BlockSpec((1, T, D), lambda b: (b, 0, 0)),
            pl.BlockSpec((1, 1), lambda b: (0, 0)),
        ],
        out_shape=[
            jax.ShapeDtypeStruct((B, T, D), jnp.float32),
            jax.ShapeDtypeStruct((1, 1), jnp.float32),
        ],
        scratch_shapes=[pltpu.SMEM((1, 1), jnp.float32)],
        compiler_params=pltpu.CompilerParams(
            dimension_semantics=("arbitrary",),
        ),
    )(zq3, zn3)


def _sc_gather(Wp, idx):
    info = plsc.get_sparse_core_info()
    nc, ns = info.num_cores, info.num_subcores
    nw = nc * ns
    n = idx.shape[0]
    b_per_w = n // nw
    chunks = b_per_w // IDX_CHUNK
    mesh = plsc.VectorSubcoreMesh(core_axis_name="c", subcore_axis_name="s")

    @functools.partial(
        pl.kernel,
        mesh=mesh,
        out_type=jax.ShapeDtypeStruct((n, DP), jnp.float32),
        scratch_types=[
            pltpu.VMEM((IDX_CHUNK,), jnp.int32),
            pltpu.VMEM((IDX_CHUNK, DP), jnp.float32),
            pltpu.SemaphoreType.DMA,
        ],
    )
    def gather_kernel(table_hbm, idx_hbm, out_hbm, idx_v, rows_v, sem):
        wid = lax.axis_index("s") * nc + lax.axis_index("c")
        base = wid * b_per_w
        for j in range(chunks):
            off = base + j * IDX_CHUNK
            pltpu.sync_copy(idx_hbm.at[pl.ds(off, IDX_CHUNK)], idx_v)
            pltpu.async_copy(table_hbm.at[idx_v], rows_v, sem).wait()
            pltpu.sync_copy(rows_v, out_hbm.at[pl.ds(off, IDX_CHUNK)])

    return gather_kernel(Wp, idx)


def kernel(z, W):
    nz = jnp.linalg.norm(z, ord=2, axis=1, keepdims=True)
    zn3 = z / jnp.maximum(nz, 1e-12)
    nw_ = jnp.linalg.norm(W, ord=2, axis=1, keepdims=True)
    en = W / jnp.maximum(nw_, 1e-12)
    z_flat = zn3.reshape(B * T, D)
    a = jnp.sum(z_flat ** 2, axis=1, keepdims=True)
    a3 = a.reshape(B, T)[:, None, :]
    c2 = jnp.sum(W ** 2, axis=1).reshape(K, 1)
    znb3 = (zn3 * 2.0).astype(jnp.bfloat16)
    enb = en.astype(jnp.bfloat16)

    idx3 = _argmin_call(znb3, enb, a3, c2)
    idx = idx3.reshape(B * T)

    Wp = jnp.pad(W, ((0, 0), (0, DP - D)))
    rows = _sc_gather(Wp, idx)
    zq3 = rows.reshape(B, T, DP)

    zq_out, loss11 = _finalize_call(zq3, zn3)
    return zq_out, idx, loss11[0, 0]


# speedup vs baseline: 1.3094x; 1.0378x over previous
import functools

import jax
import jax.numpy as jnp
from jax import lax
from jax.experimental import pallas as pl
from jax.experimental.pallas import tpu as pltpu
from jax.experimental.pallas import tpu_sc as plsc

B, T, D = 16, 1024, 32
DP = 128
K = 8192
KT = 2048
KSTEPS = K // KT
CHUNK = 4096
TILES_PER_CHUNK = CHUNK // KT
IDX_CHUNK = 128


def _argmin_body(zn_ref, en_ref, a_ref, c_ref, out_ref, dw_ref, cm_ref, ci_ref, mn_ref, ix_ref):
    k = pl.program_id(1)
    znb = zn_ref[0]
    enb = en_ref[...]
    s2 = lax.dot_general(
        enb, znb, (((1,), (1,)), ((), ())),
        preferred_element_type=jnp.float32,
    )
    dist = (a_ref[0] + c_ref[...]) - s2
    dw_ref[pl.ds((k % TILES_PER_CHUNK) * KT, KT), :] = dist
    m = jnp.min(dist, axis=0, keepdims=True)

    @pl.when(k % TILES_PER_CHUNK == 0)
    def _():
        cm_ref[...] = m

    @pl.when(k % TILES_PER_CHUNK != 0)
    def _():
        cm_ref[...] = jnp.minimum(m, cm_ref[...])

    @pl.when(k % TILES_PER_CHUNK == TILES_PER_CHUNK - 1)
    def _():
        cm = cm_ref[...]
        dw = dw_ref[...]
        ids = lax.broadcasted_iota(jnp.int32, dw.shape, 0)
        ci = jnp.min(
            jnp.where(dw == cm, ids, jnp.int32(2**31 - 1)), axis=0, keepdims=True
        ) + (k // TILES_PER_CHUNK) * CHUNK
        cmb = cm.astype(jnp.bfloat16).astype(jnp.float32)

        @pl.when(k == TILES_PER_CHUNK - 1)
        def _():
            mn_ref[...] = cmb
            ix_ref[...] = ci

        @pl.when(k > TILES_PER_CHUNK - 1)
        def _():
            win = mn_ref[...] > cm
            mn_ref[...] = jnp.where(win, cmb, mn_ref[...])
            ix_ref[...] = jnp.where(win, ci, ix_ref[...])

    @pl.when(k == KSTEPS - 1)
    def _():
        out_ref[0] = ix_ref[...]


def _argmin_call(zn3, en, a3, c2):
    return pl.pallas_call(
        _argmin_body,
        grid=(B, KSTEPS),
        in_specs=[
            pl.BlockSpec((1, T, D), lambda b, k: (b, 0, 0)),
            pl.BlockSpec((KT, D), lambda b, k: (k, 0)),
            pl.BlockSpec((1, 1, T), lambda b, k: (b, 0, 0)),
            pl.BlockSpec((KT, 1), lambda b, k: (k, 0)),
        ],
        out_specs=pl.BlockSpec((1, 1, T), lambda b, k: (b, 0, 0)),
        out_shape=jax.ShapeDtypeStruct((B, 1, T), jnp.int32),
        scratch_shapes=[
            pltpu.VMEM((CHUNK, T), jnp.float32),
            pltpu.VMEM((1, T), jnp.float32),
            pltpu.VMEM((1, T), jnp.int32),
            pltpu.VMEM((1, T), jnp.float32),
            pltpu.VMEM((1, T), jnp.int32),
        ],
        compiler_params=pltpu.CompilerParams(
            dimension_semantics=("arbitrary", "arbitrary"),
        ),
    )(zn3, en, a3, c2)


def _finalize_body(zq_ref, zn_ref, out_ref, loss_ref, acc_ref):
    b = pl.program_id(0)
    zq = zq_ref[0][:, :D]
    znb = zn_ref[0]
    n = jnp.sqrt(jnp.sum(zq * zq, axis=0, keepdims=True))
    zqn = zq / jnp.maximum(n, 1e-12)
    out_ref[0] = znb + (zqn - znb)
    part = jnp.sum((zqn - znb) ** 2)

    @pl.when(b == 0)
    def _():
        acc_ref[0, 0] = part

    @pl.when(b > 0)
    def _():
        acc_ref[0, 0] = acc_ref[0, 0] + part

    @pl.when(b == B - 1)
    def _():
        m = acc_ref[0, 0] / (B * T * D)
        loss_ref[...] = jnp.full((1, 1), m + 0.25 * m, jnp.float32)


def _finalize_call(zq3, zn3):
    return pl.pallas_call(
        _finalize_body,
        grid=(B,),
        in_specs=[
            pl.BlockSpec((1, T, DP), lambda b: (b, 0, 0)),
            pl.BlockSpec((1, T, D), lambda b: (b, 0, 0)),
        ],
        out_specs=[
            pl.BlockSpec((1, T, D), lambda b: (b, 0, 0)),
            pl.BlockSpec((1, 1), lambda b: (0, 0)),
        ],
        out_shape=[
            jax.ShapeDtypeStruct((B, T, D), jnp.float32),
            jax.ShapeDtypeStruct((1, 1), jnp.float32),
        ],
        scratch_shapes=[pltpu.SMEM((1, 1), jnp.float32)],
        compiler_params=pltpu.CompilerParams(
            dimension_semantics=("arbitrary",),
        ),
    )(zq3, zn3)


def _sc_gather(Wp, idx):
    info = plsc.get_sparse_core_info()
    nc, ns = info.num_cores, info.num_subcores
    nw = nc * ns
    n = idx.shape[0]
    b_per_w = n // nw
    chunks = b_per_w // IDX_CHUNK
    mesh = plsc.VectorSubcoreMesh(core_axis_name="c", subcore_axis_name="s")

    @functools.partial(
        pl.kernel,
        mesh=mesh,
        out_type=jax.ShapeDtypeStruct((n, DP), jnp.float32),
        scratch_types=[
            pltpu.VMEM((IDX_CHUNK,), jnp.int32),
            pltpu.VMEM((IDX_CHUNK, DP), jnp.float32),
            pltpu.SemaphoreType.DMA,
        ],
    )
    def gather_kernel(table_hbm, idx_hbm, out_hbm, idx_v, rows_v, sem):
        wid = lax.axis_index("s") * nc + lax.axis_index("c")
        base = wid * b_per_w
        for j in range(chunks):
            off = base + j * IDX_CHUNK
            pltpu.sync_copy(idx_hbm.at[pl.ds(off, IDX_CHUNK)], idx_v)
            pltpu.async_copy(table_hbm.at[idx_v], rows_v, sem).wait()
            pltpu.sync_copy(rows_v, out_hbm.at[pl.ds(off, IDX_CHUNK)])

    return gather_kernel(Wp, idx)


def kernel(z, W):
    nz = jnp.linalg.norm(z, ord=2, axis=1, keepdims=True)
    zn3 = z / jnp.maximum(nz, 1e-12)
    nw_ = jnp.linalg.norm(W, ord=2, axis=1, keepdims=True)
    en = W / jnp.maximum(nw_, 1e-12)
    z_flat = zn3.reshape(B * T, D)
    a = jnp.sum(z_flat ** 2, axis=1, keepdims=True)
    a3 = a.reshape(B, T)[:, None, :]
    c2 = jnp.sum(W ** 2, axis=1).reshape(K, 1)
    znb3 = (zn3 * 2.0).astype(jnp.bfloat16)
    enb = en.astype(jnp.bfloat16)

    idx3 = _argmin_call(znb3, enb, a3, c2)
    idx = idx3.reshape(B * T)

    Wp = jnp.pad(W, ((0, 0), (0, DP - D)))
    rows = _sc_gather(Wp, idx)
    zq3 = rows.reshape(B, T, DP)

    zq_out, loss11 = _finalize_call(zq3, zn3)
    return zq_out, idx, loss11[0, 0]
